# baseline (speedup 1.0000x reference)
import functools

import jax
import jax.numpy as jnp
from jax import lax
from jax.experimental import pallas as pl
from jax.experimental.pallas import tpu as pltpu
from jax.experimental.pallas import tpu_sc as plsc

B, H, W, C = 4, 112, 112, 96
OH, OW = 2 * H, 2 * W
HWC = H * W * C
OHOWC = OH * OW * C
NREG = 3
ROWW = OW * C
REG_ROWS = (75, 75, 74)
REG_BASE = (0, 75, 150)
RMAX = 75 * ROWW
TRASH = 12288
ACC = RMAX + TRASH
NS = 16
PT = HWC // NS
WIN = 2688
NWIN = PT // WIN
ZCH = 4416
NZ = (ACC // NS) // ZCH
NVR = WIN // 16


def _body(x_hbm, idx_hbm, out_hbm,
          idx0, idx1, idx2, val0, val1, val2, dst0, dst1, dst2, zbuf, acc,
          zsem, lsem0, lsem1, lsem2, ssem0, ssem1, ssem2, wsem):
    core = lax.axis_index("c")
    sid = lax.axis_index("s")
    lane = lax.iota(jnp.int32, 16)
    idx_bufs = (idx0, idx1, idx2)
    val_bufs = (val0, val1, val2)
    dst_bufs = (dst0, dst1, dst2)
    lsems = (lsem0, lsem1, lsem2)
    ssems = (ssem0, ssem1, ssem2)
    c96 = jnp.full((16,), 96, jnp.int32)
    third = jnp.full((16,), 1.0 / 3.0, jnp.float32)
    five = jnp.full((16,), 5, jnp.int32)

    def zfill(i, carry):
        zbuf[pl.ds(i * 16, 16)] = jnp.zeros((16,), jnp.float32)
        return carry

    lax.fori_loop(0, ZCH // 16, zfill, 0)

    for u in range(2 * NREG):
        b = core * 2 + (u // NREG)
        r = u % NREG
        rbase = REG_BASE[r] * ROWW
        nrows = REG_ROWS[r]
        rsize = nrows * ROWW
        ru32 = jnp.full((16,), rsize, jnp.uint32)

        z0 = sid * (ACC // NS)

        def zs(i, carry):
            pltpu.async_copy(zbuf, acc.at[pl.ds(z0 + i * ZCH, ZCH)], zsem)
            return carry

        lax.fori_loop(0, NZ, zs, 0)

        def zw(i, carry):
            pltpu.make_async_copy(zbuf, acc.at[pl.ds(z0, ZCH)], zsem).wait()
            return carry

        lax.fori_loop(0, NZ, zw, 0)
        plsc.subcore_barrier()

        el_t0 = b * HWC + sid * PT

        def start_load(w, p):
            e0 = el_t0 + w * WIN
            pltpu.async_copy(idx_hbm.at[pl.ds(e0, WIN)], idx_bufs[p],
                             lsems[p])
            pltpu.async_copy(x_hbm.at[pl.ds(e0, WIN)], val_bufs[p], lsems[p])

        def wait_load(p):
            pltpu.make_async_copy(idx_hbm.at[pl.ds(0, WIN)], idx_bufs[p],
                                  lsems[p]).wait()
            pltpu.make_async_copy(x_hbm.at[pl.ds(0, WIN)], val_bufs[p],
                                  lsems[p]).wait()

        def start_scatter(p):
            pltpu.async_copy(val_bufs[p], acc.at[dst_bufs[p]], ssems[p],
                             add=True)

        def wait_scatter(p):
            pltpu.make_async_copy(val_bufs[p], acc.at[dst_bufs[p]],
                                  ssems[p]).wait()

        tvec0 = lane + RMAX + sid * 768

        def compute(w, p):
            ib = idx_bufs[p]
            db = dst_bufs[p]

            def jbody(j6, carry2):
                for jj in range(6):
                    j = j6 * 6 + jj
                    iv = ib[pl.ds(j * 16, 16)]
                    m = lax.shift_right_logical(iv, five)
                    q = lax.convert_element_type(
                        lax.convert_element_type(m, jnp.float32) * third,
                        jnp.int32)
                    cr = lane + ((16 * jj) % 96 - rbase)
                    loc = q * c96 + cr
                    ok = lax.lt(lax.bitcast_convert_type(loc, jnp.uint32),
                                ru32)
                    trash = tvec0 + lax.broadcast_in_dim(
                        ((j % 48) * 16).astype(jnp.int32), (16,), ())
                    db[pl.ds(j * 16, 16)] = lax.select(ok, loc, trash)
                return carry2

            lax.fori_loop(0, NVR // 6, jbody, 0)

        def window(w, p):
            @pl.when(jnp.logical_and(w >= 2, w + 1 < NWIN))
            def _():
                wait_scatter((p + 1) % 3)

            @pl.when(w + 1 < NWIN)
            def _():
                start_load(w + 1, (p + 1) % 3)

            wait_load(p)
            compute(w, p)
            start_scatter(p)

        start_load(0, 0)

        def wloop(t, carry):
            w0 = 3 * t
            window(w0, 0)
            window(w0 + 1, 1)
            window(w0 + 2, 2)
            return carry

        lax.fori_loop(0, NWIN // 3, wloop, 0)
        window(NWIN - 1, 0)
        wait_scatter(1)
        wait_scatter(2)
        wait_scatter(0)
        plsc.subcore_barrier()

        for i in range(5):
            @pl.when(sid + 16 * i < nrows)
            def _():
                row = sid + 16 * i
                pltpu.async_copy(acc.at[pl.ds(row * ROWW, ROWW)],
                                 out_hbm.at[b, REG_BASE[r] + row], wsem)
        for i in range(5):
            @pl.when(sid + 16 * i < nrows)
            def _():
                pltpu.make_async_copy(acc.at[pl.ds(0, ROWW)],
                                      out_hbm.at[b, REG_BASE[r]],
                                      wsem).wait()


_sc_call = functools.partial(
    pl.kernel,
    out_type=jax.ShapeDtypeStruct((B, OH, ROWW), jnp.float32),
    mesh=plsc.VectorSubcoreMesh(core_axis_name="c", subcore_axis_name="s"),
    scratch_types=[
        pltpu.VMEM((WIN,), jnp.int32),
        pltpu.VMEM((WIN,), jnp.int32),
        pltpu.VMEM((WIN,), jnp.int32),
        pltpu.VMEM((WIN,), jnp.float32),
        pltpu.VMEM((WIN,), jnp.float32),
        pltpu.VMEM((WIN,), jnp.float32),
        pltpu.VMEM((WIN,), jnp.int32),
        pltpu.VMEM((WIN,), jnp.int32),
        pltpu.VMEM((WIN,), jnp.int32),
        pltpu.VMEM((ZCH,), jnp.float32),
        pltpu.VMEM_SHARED((ACC,), jnp.float32),
        pltpu.SemaphoreType.DMA,
        pltpu.SemaphoreType.DMA,
        pltpu.SemaphoreType.DMA,
        pltpu.SemaphoreType.DMA,
        pltpu.SemaphoreType.DMA,
        pltpu.SemaphoreType.DMA,
        pltpu.SemaphoreType.DMA,
        pltpu.SemaphoreType.DMA,
    ],
)(_body)


@jax.jit
def kernel(x, pool_indices):
    idx = pool_indices.astype(jnp.int32).reshape(B * HWC)
    xf = x.reshape(B * HWC)
    out = _sc_call(xf, idx)
    return out.reshape(B, OH, OW, C)


# speedup vs baseline: 1.4034x; 1.4034x over previous
import functools

import jax
import jax.numpy as jnp
from jax import lax
from jax.experimental import pallas as pl
from jax.experimental.pallas import tpu as pltpu
from jax.experimental.pallas import tpu_sc as plsc

B, H, W, C = 4, 112, 112, 96
OH, OW = 2 * H, 2 * W
HWC = H * W * C
OHOWC = OH * OW * C
NREG = 3
R = OHOWC // NREG
TRASH = 32768
ACC = R + TRASH
NS = 16
PT = HWC // NS
WIN = 2688
NWIN = PT // WIN
STRIPE = R // NS
ZCH = 3200
NZ = (ACC // NS) // ZCH
NVR = WIN // 16


def _body(x_hbm, idx_hbm, out_hbm,
          idx0, idx1, idx2, val0, val1, val2, dst0, dst1, dst2, zbuf, acc,
          zsem, lsem0, lsem1, lsem2, ssem0, ssem1, ssem2):
    core = lax.axis_index("c")
    sid = lax.axis_index("s")
    lane = lax.iota(jnp.int32, 16)
    idx_bufs = (idx0, idx1, idx2)
    val_bufs = (val0, val1, val2)
    dst_bufs = (dst0, dst1, dst2)
    lsems = (lsem0, lsem1, lsem2)
    ssems = (ssem0, ssem1, ssem2)
    c96 = jnp.full((16,), 96, jnp.int32)
    third = jnp.full((16,), 1.0 / 3.0, jnp.float32)
    five = jnp.full((16,), 5, jnp.int32)
    ru32 = jnp.full((16,), R, jnp.uint32)

    def zfill(i, carry):
        zbuf[pl.ds(i * 16, 16)] = jnp.zeros((16,), jnp.float32)
        return carry

    lax.fori_loop(0, ZCH // 16, zfill, 0)

    for u in range(2 * NREG):
        b = core * 2 + (u // NREG)
        r = u % NREG
        rbase = r * R

        el_t0 = b * HWC + sid * PT

        def start_load(w, p):
            e0 = el_t0 + w * WIN
            pltpu.async_copy(idx_hbm.at[pl.ds(e0, WIN)], idx_bufs[p],
                             lsems[p])
            pltpu.async_copy(x_hbm.at[pl.ds(e0, WIN)], val_bufs[p], lsems[p])

        def start_load0():
            start_load(0, 0)

        z0 = sid * (ACC // NS)

        def zs(i, carry):
            pltpu.async_copy(zbuf, acc.at[pl.ds(z0 + i * ZCH, ZCH)], zsem)
            return carry

        start_load0()
        lax.fori_loop(0, NZ, zs, 0)

        def zw(i, carry):
            pltpu.make_async_copy(zbuf, acc.at[pl.ds(z0, ZCH)], zsem).wait()
            return carry

        lax.fori_loop(0, NZ, zw, 0)
        plsc.subcore_barrier()

        def wait_load(p):
            pltpu.make_async_copy(idx_hbm.at[pl.ds(0, WIN)], idx_bufs[p],
                                  lsems[p]).wait()
            pltpu.make_async_copy(x_hbm.at[pl.ds(0, WIN)], val_bufs[p],
                                  lsems[p]).wait()

        def start_scatter(p):
            pltpu.async_copy(val_bufs[p], acc.at[dst_bufs[p]], ssems[p],
                             add=True)

        def wait_scatter(p):
            pltpu.make_async_copy(val_bufs[p], acc.at[dst_bufs[p]],
                                  ssems[p]).wait()

        tvec0 = lane + R + sid * 2048

        def compute(w, p):
            ib = idx_bufs[p]
            db = dst_bufs[p]

            def jbody(j6, carry2):
                for jj in range(6):
                    j = j6 * 6 + jj
                    iv = ib[pl.ds(j * 16, 16)]
                    m = lax.shift_right_logical(iv, five)
                    q = lax.convert_element_type(
                        lax.convert_element_type(m, jnp.float32) * third,
                        jnp.int32)
                    cr = lane + ((16 * jj) % 96 - rbase)
                    loc = q * c96 + cr
                    ok = lax.lt(lax.bitcast_convert_type(loc, jnp.uint32),
                                ru32)
                    trash = tvec0 + lax.broadcast_in_dim(
                        ((j % 128) * 16).astype(jnp.int32), (16,), ())
                    db[pl.ds(j * 16, 16)] = lax.select(ok, loc, trash)
                return carry2

            lax.fori_loop(0, NVR // 6, jbody, 0)

        def window(w, p):
            @pl.when(jnp.logical_and(w >= 2, w + 1 < NWIN))
            def _():
                wait_scatter((p + 1) % 3)

            @pl.when(w + 1 < NWIN)
            def _():
                start_load(w + 1, (p + 1) % 3)

            wait_load(p)
            compute(w, p)
            start_scatter(p)

        def wloop(t, carry):
            w0 = 3 * t
            window(w0, 0)
            window(w0 + 1, 1)
            window(w0 + 2, 2)
            return carry

        lax.fori_loop(0, NWIN // 3, wloop, 0)
        window(NWIN - 1, 0)
        wait_scatter(1)
        wait_scatter(2)
        wait_scatter(0)
        plsc.subcore_barrier()

        ob = b * OHOWC + rbase + sid * STRIPE
        pltpu.sync_copy(acc.at[pl.ds(sid * STRIPE, STRIPE)],
                        out_hbm.at[pl.ds(ob, STRIPE)])


_sc_call = functools.partial(
    pl.kernel,
    out_type=jax.ShapeDtypeStruct((B * OHOWC,), jnp.float32),
    mesh=plsc.VectorSubcoreMesh(core_axis_name="c", subcore_axis_name="s"),
    scratch_types=[
        pltpu.VMEM((WIN,), jnp.int32),
        pltpu.VMEM((WIN,), jnp.int32),
        pltpu.VMEM((WIN,), jnp.int32),
        pltpu.VMEM((WIN,), jnp.float32),
        pltpu.VMEM((WIN,), jnp.float32),
        pltpu.VMEM((WIN,), jnp.float32),
        pltpu.VMEM((WIN,), jnp.int32),
        pltpu.VMEM((WIN,), jnp.int32),
        pltpu.VMEM((WIN,), jnp.int32),
        pltpu.VMEM((ZCH,), jnp.float32),
        pltpu.VMEM_SHARED((ACC,), jnp.float32),
        pltpu.SemaphoreType.DMA,
        pltpu.SemaphoreType.DMA,
        pltpu.SemaphoreType.DMA,
        pltpu.SemaphoreType.DMA,
        pltpu.SemaphoreType.DMA,
        pltpu.SemaphoreType.DMA,
        pltpu.SemaphoreType.DMA,
    ],
)(_body)


@jax.jit
def kernel(x, pool_indices):
    idx = pool_indices.astype(jnp.int32).reshape(B * HWC)
    xf = x.reshape(B * HWC)
    out = _sc_call(xf, idx)
    return out.reshape(B, OH, OW, C)


# speedup vs baseline: 1.4050x; 1.0011x over previous
import functools

import jax
import jax.numpy as jnp
from jax import lax
from jax.experimental import pallas as pl
from jax.experimental.pallas import tpu as pltpu
from jax.experimental.pallas import tpu_sc as plsc

B, H, W, C = 4, 112, 112, 96
OH, OW = 2 * H, 2 * W
HWC = H * W * C
OHOWC = OH * OW * C
NREG = 3
R = OHOWC // NREG
TRASH = 32768
ACC = R + TRASH
NS = 16
PT = HWC // NS
WIN = 2688
NWIN = PT // WIN
STRIPE = R // NS
ZCH = 3200
NZ = (ACC // NS) // ZCH
NVR = WIN // 16


def _body(x_hbm, idx_hbm, out_hbm,
          idx0, idx1, idx2, val0, val1, val2, dst0, dst1, dst2, zbuf, acc,
          zsem, lsem0, lsem1, lsem2, ssem0, ssem1, ssem2):
    core = lax.axis_index("c")
    sid = lax.axis_index("s")
    lane = lax.iota(jnp.int32, 16)
    idx_bufs = (idx0, idx1, idx2)
    val_bufs = (val0, val1, val2)
    dst_bufs = (dst0, dst1, dst2)
    lsems = (lsem0, lsem1, lsem2)
    ssems = (ssem0, ssem1, ssem2)
    c96 = jnp.full((16,), 96, jnp.int32)
    third = jnp.full((16,), 1.0 / 3.0, jnp.float32)
    five = jnp.full((16,), 5, jnp.int32)
    ru32 = jnp.full((16,), R, jnp.uint32)

    def zfill(i, carry):
        zbuf[pl.ds(i * 16, 16)] = jnp.zeros((16,), jnp.float32)
        return carry

    lax.fori_loop(0, ZCH // 16, zfill, 0)

    for u in range(2 * NREG):
        b = core * 2 + (u // NREG)
        r = u % NREG
        rbase = r * R

        el_t0 = b * HWC + sid * PT

        def start_load(w, p):
            e0 = el_t0 + w * WIN
            pltpu.async_copy(idx_hbm.at[pl.ds(e0, WIN)], idx_bufs[p],
                             lsems[p])
            pltpu.async_copy(x_hbm.at[pl.ds(e0, WIN)], val_bufs[p], lsems[p])

        z0 = sid * (ACC // NS)

        def zs(i, carry):
            pltpu.async_copy(zbuf, acc.at[pl.ds(z0 + i * ZCH, ZCH)], zsem)
            return carry

        start_load(0, 0)
        lax.fori_loop(0, NZ, zs, 0)

        def zw(i, carry):
            pltpu.make_async_copy(zbuf, acc.at[pl.ds(z0, ZCH)], zsem).wait()
            return carry

        lax.fori_loop(0, NZ, zw, 0)
        plsc.subcore_barrier()

        def wait_load(p):
            pltpu.make_async_copy(idx_hbm.at[pl.ds(0, WIN)], idx_bufs[p],
                                  lsems[p]).wait()
            pltpu.make_async_copy(x_hbm.at[pl.ds(0, WIN)], val_bufs[p],
                                  lsems[p]).wait()

        def start_scatter(p):
            pltpu.async_copy(val_bufs[p], acc.at[dst_bufs[p]], ssems[p],
                             add=True)

        def wait_scatter(p):
            pltpu.make_async_copy(val_bufs[p], acc.at[dst_bufs[p]],
                                  ssems[p]).wait()

        tvec0 = lane + R + sid * 2048

        def compute(w, p):
            ib = idx_bufs[p]
            db = dst_bufs[p]

            def jbody(j6, carry2):
                for jj in range(6):
                    j = j6 * 6 + jj
                    iv = ib[pl.ds(j * 16, 16)]
                    m = lax.shift_right_logical(iv, five)
                    q = lax.convert_element_type(
                        lax.convert_element_type(m, jnp.float32) * third,
                        jnp.int32)
                    cr = lane + ((16 * jj) % 96 - rbase)
                    loc = q * c96 + cr
                    ok = lax.lt(lax.bitcast_convert_type(loc, jnp.uint32),
                                ru32)
                    trash = tvec0 + lax.broadcast_in_dim(
                        ((j % 128) * 16).astype(jnp.int32), (16,), ())
                    db[pl.ds(j * 16, 16)] = lax.select(ok, loc, trash)
                return carry2

            lax.fori_loop(0, NVR // 6, jbody, 0)

        def window(w, p):
            @pl.when(jnp.logical_and(w >= 2, w + 1 < NWIN))
            def _():
                wait_scatter((p + 1) % 3)

            @pl.when(w + 1 < NWIN)
            def _():
                start_load(w + 1, (p + 1) % 3)

            wait_load(p)
            compute(w, p)
            start_scatter(p)

        def wloop(t, carry):
            w0 = 3 * t
            window(w0, 0)
            window(w0 + 1, 1)
            window(w0 + 2, 2)
            return carry

        lax.fori_loop(0, NWIN // 3, wloop, 0)
        window(NWIN - 1, 0)
        wait_scatter(1)
        wait_scatter(2)
        wait_scatter(0)
        plsc.subcore_barrier()

        ob = b * OHOWC + rbase + sid * STRIPE
        pltpu.sync_copy(acc.at[pl.ds(sid * STRIPE, STRIPE)],
                        out_hbm.at[pl.ds(ob, STRIPE)])


_sc_call = functools.partial(
    pl.kernel,
    out_type=jax.ShapeDtypeStruct((B * OHOWC,), jnp.float32),
    mesh=plsc.VectorSubcoreMesh(core_axis_name="c", subcore_axis_name="s"),
    scratch_types=[
        pltpu.VMEM((WIN,), jnp.int32),
        pltpu.VMEM((WIN,), jnp.int32),
        pltpu.VMEM((WIN,), jnp.int32),
        pltpu.VMEM((WIN,), jnp.float32),
        pltpu.VMEM((WIN,), jnp.float32),
        pltpu.VMEM((WIN,), jnp.float32),
        pltpu.VMEM((WIN,), jnp.int32),
        pltpu.VMEM((WIN,), jnp.int32),
        pltpu.VMEM((WIN,), jnp.int32),
        pltpu.VMEM((ZCH,), jnp.float32),
        pltpu.VMEM_SHARED((ACC,), jnp.float32),
        pltpu.SemaphoreType.DMA,
        pltpu.SemaphoreType.DMA,
        pltpu.SemaphoreType.DMA,
        pltpu.SemaphoreType.DMA,
        pltpu.SemaphoreType.DMA,
        pltpu.SemaphoreType.DMA,
        pltpu.SemaphoreType.DMA,
    ],
)(_body)


@jax.jit
def kernel(x, pool_indices):
    idx = pool_indices.astype(jnp.int32).reshape(B * HWC)
    xf = x.reshape(B * HWC)
    out = _sc_call(xf, idx)
    return out.reshape(B, OH, OW, C)
